# baseline (device time: 32145 ns/iter reference)
import jax
import jax.numpy as jnp
from jax import lax
from jax.experimental import pallas as pl
from jax.experimental.pallas import tpu as pltpu

N_DEV = 4
NSUB = 2
NHOP = N_DEV - 2
NRS = N_DEV - 1


def kernel(x, W1, W2):
    m, d = x.shape
    f = W1.shape[1]
    M = N_DEV * m
    q = m // 2
    s = q // NSUB

    def body(x_ref, w1_ref, w2_ref, out_ref,
             agcR_ref, agcL_ref, dirA_ref, dirB_ref,
             w1f_ref, w2f_ref, w1b_ref, w2b_ref, acc_ref,
             rscR_ref, rscL_ref, w_sems, dir_sems,
             agR_send, agR_recv, agL_send, agL_recv,
             rsR_send, rsR_recv, rsL_send, rsL_recv):
        my = lax.axis_index("i")
        left = (my - 1) % N_DEV
        right = (my + 1) % N_DEV

        w1_copy = pltpu.make_async_copy(w1_ref, w1f_ref, w_sems.at[0])
        w2_copy = pltpu.make_async_copy(w2_ref, w2f_ref, w_sems.at[1])
        w1_copy.start()
        w2_copy.start()

        def copy(src, dst, ssem, rsem, target):
            return pltpu.make_async_remote_copy(
                src_ref=src, dst_ref=dst, send_sem=ssem, recv_sem=rsem,
                device_id=(target,), device_id_type=pl.DeviceIdType.MESH,
            )

        def compute_sub(origin, src_slot, is_b, j):
            hblk = jnp.dot(src_slot[:, :], w1b_ref[:, :],
                           preferred_element_type=jnp.float32
                           ).astype(jnp.bfloat16)
            one = jnp.bfloat16(1.0)
            hblk = hblk * (one / (one + jnp.exp(-hblk)))
            row0 = origin * m + (q if is_b else 0) + j * s
            acc_ref[pl.ds(row0, s), :] = jnp.dot(
                hblk, w2b_ref[:, :],
                preferred_element_type=jnp.float32)

        def acc_sub(b, is_b, j):
            return acc_ref[pl.ds(b * m + (q if is_b else 0) + j * s, s), :]

        barrier_sem = pltpu.get_barrier_semaphore()
        for nbr in (left, right):
            pl.semaphore_signal(
                barrier_sem, inc=1,
                device_id=(nbr,), device_id_type=pl.DeviceIdType.MESH,
            )
        pl.semaphore_wait(barrier_sem, 2)

        agcR_ref[0, :, :] = x_ref[pl.ds(0, q), :].astype(jnp.bfloat16)
        agcL_ref[0, :, :] = x_ref[pl.ds(q, q), :].astype(jnp.bfloat16)
        agR = [[None] * NSUB for _ in range(NHOP)]
        agL = [[None] * NSUB for _ in range(NHOP)]
        rsR = [[None] * NSUB for _ in range(NRS)]
        rsL = [[None] * NSUB for _ in range(NRS)]
        dirA = copy(agcR_ref.at[0], dirA_ref,
                    dir_sems.at[0], dir_sems.at[1], left)
        dirB = copy(agcL_ref.at[0], dirB_ref,
                    dir_sems.at[2], dir_sems.at[3], right)
        dirA.start()
        dirB.start()
        for j in range(NSUB):
            agR[0][j] = copy(agcR_ref.at[0, pl.ds(j * s, s)],
                             agcR_ref.at[1, pl.ds(j * s, s)],
                             agR_send.at[0, j], agR_recv.at[0, j], right)
            agR[0][j].start()
            agL[0][j] = copy(agcL_ref.at[0, pl.ds(j * s, s)],
                             agcL_ref.at[1, pl.ds(j * s, s)],
                             agL_send.at[0, j], agL_recv.at[0, j], left)
            agL[0][j].start()

        w1_copy.wait()
        w1b_ref[:, :] = w1f_ref[:, :].astype(jnp.bfloat16)
        w2_copy.wait()
        w2b_ref[:, :] = w2f_ref[:, :].astype(jnp.bfloat16)
        for j in range(NSUB):
            compute_sub(my, agcR_ref.at[0, pl.ds(j * s, s)], False, j)
            compute_sub(my, agcL_ref.at[0, pl.ds(j * s, s)], True, j)

        def ring_step(h, j, ag, agc_ref, ag_send, ag_recv,
                      rs, rsc_ref, rs_send, rs_recv, origin, is_b, target):
            sub = pl.ds(j * s, s)
            ag[h][j].wait_recv()
            if h + 1 < NHOP:
                ag[h + 1][j] = copy(agc_ref.at[h + 1, sub],
                                    agc_ref.at[h + 2, sub],
                                    ag_send.at[h + 1, j],
                                    ag_recv.at[h + 1, j], target)
                ag[h + 1][j].start()
            compute_sub(origin, agc_ref.at[h + 1, sub], is_b, j)
            if h == 0:
                rsc_ref[0, sub] = acc_sub(origin, is_b, j).astype(jnp.bfloat16)
            else:
                rs[h - 1][j].wait_recv()
                rsc_ref[h, sub] = (
                    rsc_ref[h, sub].astype(jnp.float32)
                    + acc_sub(origin, is_b, j)
                ).astype(jnp.bfloat16)
            rs[h][j] = copy(rsc_ref.at[h, sub], rsc_ref.at[h + 1, sub],
                            rs_send.at[h, j], rs_recv.at[h, j], target)
            rs[h][j].start()

        for h in range(NHOP):
            oR = (my - h - 1) % N_DEV
            oL = (my + h + 1) % N_DEV
            for j in range(NSUB):
                ring_step(h, j, agR, agcR_ref, agR_send, agR_recv,
                          rsR, rscR_ref, rsR_send, rsR_recv, oR, False, right)
                ring_step(h, j, agL, agcL_ref, agL_send, agL_recv,
                          rsL, rscL_ref, rsL_send, rsL_recv, oL, True, left)

        dirA.wait_recv()
        dirB.wait_recv()
        tA = (my + 1) % N_DEV
        tB = (my - 1) % N_DEV
        for j in range(NSUB):
            compute_sub(tA, dirA_ref.at[pl.ds(j * s, s)], False, j)
            compute_sub(tB, dirB_ref.at[pl.ds(j * s, s)], True, j)

        hL = NRS - 1
        for j in range(NSUB):
            sub = pl.ds(j * s, s)
            rsR[hL - 1][j].wait_recv()
            rscR_ref[hL, sub] = (
                rscR_ref[hL, sub].astype(jnp.float32) + acc_sub(tA, False, j)
            ).astype(jnp.bfloat16)
            rsR[hL][j] = copy(rscR_ref.at[hL, sub], rscR_ref.at[hL + 1, sub],
                              rsR_send.at[hL, j], rsR_recv.at[hL, j], right)
            rsR[hL][j].start()
            rsL[hL - 1][j].wait_recv()
            rscL_ref[hL, sub] = (
                rscL_ref[hL, sub].astype(jnp.float32) + acc_sub(tB, True, j)
            ).astype(jnp.bfloat16)
            rsL[hL][j] = copy(rscL_ref.at[hL, sub], rscL_ref.at[hL + 1, sub],
                              rsL_send.at[hL, j], rsL_recv.at[hL, j], left)
            rsL[hL][j].start()

        for j in range(NSUB):
            sub = pl.ds(j * s, s)
            rsR[hL][j].wait_recv()
            out_ref[pl.ds(j * s, s), :] = (
                rscR_ref[NRS, sub].astype(jnp.float32) + acc_sub(my, False, j)
            )
            rsL[hL][j].wait_recv()
            out_ref[pl.ds(q + j * s, s), :] = (
                rscL_ref[NRS, sub].astype(jnp.float32) + acc_sub(my, True, j)
            )

        dirA.wait_send()
        dirB.wait_send()
        for h in range(NHOP):
            for j in range(NSUB):
                agR[h][j].wait_send()
                agL[h][j].wait_send()
        for h in range(NRS):
            for j in range(NSUB):
                rsR[h][j].wait_send()
                rsL[h][j].wait_send()

    return pl.pallas_call(
        body,
        out_shape=jax.ShapeDtypeStruct((m, d), jnp.float32),
        in_specs=[
            pl.BlockSpec(memory_space=pltpu.VMEM),
            pl.BlockSpec(memory_space=pltpu.MemorySpace.HBM),
            pl.BlockSpec(memory_space=pltpu.MemorySpace.HBM),
        ],
        out_specs=pl.BlockSpec(memory_space=pltpu.VMEM),
        scratch_shapes=[
            pltpu.VMEM((NHOP + 1, q, d), jnp.bfloat16),
            pltpu.VMEM((NHOP + 1, q, d), jnp.bfloat16),
            pltpu.VMEM((q, d), jnp.bfloat16),
            pltpu.VMEM((q, d), jnp.bfloat16),
            pltpu.VMEM((d, f), jnp.float32),
            pltpu.VMEM((f, d), jnp.float32),
            pltpu.VMEM((d, f), jnp.bfloat16),
            pltpu.VMEM((f, d), jnp.bfloat16),
            pltpu.VMEM((M, d), jnp.float32),
            pltpu.VMEM((NRS + 1, q, d), jnp.bfloat16),
            pltpu.VMEM((NRS + 1, q, d), jnp.bfloat16),
            pltpu.SemaphoreType.DMA((2,)),
            pltpu.SemaphoreType.DMA((4,)),
            pltpu.SemaphoreType.DMA((NHOP, NSUB)),
            pltpu.SemaphoreType.DMA((NHOP, NSUB)),
            pltpu.SemaphoreType.DMA((NHOP, NSUB)),
            pltpu.SemaphoreType.DMA((NHOP, NSUB)),
            pltpu.SemaphoreType.DMA((NRS, NSUB)),
            pltpu.SemaphoreType.DMA((NRS, NSUB)),
            pltpu.SemaphoreType.DMA((NRS, NSUB)),
            pltpu.SemaphoreType.DMA((NRS, NSUB)),
        ],
        compiler_params=pltpu.CompilerParams(collective_id=0),
    )(x, W1, W2)


# device time: 31483 ns/iter; 1.0210x vs baseline; 1.0210x over previous
import jax
import jax.numpy as jnp
from jax import lax
from jax.experimental import pallas as pl
from jax.experimental.pallas import tpu as pltpu

N_DEV = 4
NSUB = 2
NHOP = N_DEV - 2
NRS = N_DEV - 1


def kernel(x, W1, W2):
    m, d = x.shape
    f = W1.shape[1]
    M = N_DEV * m
    q = m // 2
    s = q // NSUB

    def body(x_ref, w1_ref, w2_ref, out_ref,
             agcR_ref, agcL_ref, dirA_ref, dirB_ref,
             w1f_ref, w2f_ref, w1b_ref, w2b_ref, acc_ref,
             rscR_ref, rscL_ref, w_sems, dir_sems,
             agR_send, agR_recv, agL_send, agL_recv,
             rsR_send, rsR_recv, rsL_send, rsL_recv):
        my = lax.axis_index("i")
        left = (my - 1) % N_DEV
        right = (my + 1) % N_DEV

        w1_copy = pltpu.make_async_copy(w1_ref, w1f_ref, w_sems.at[0])
        w2_copy = pltpu.make_async_copy(w2_ref, w2f_ref, w_sems.at[1])
        w1_copy.start()
        w2_copy.start()

        def copy(src, dst, ssem, rsem, target):
            return pltpu.make_async_remote_copy(
                src_ref=src, dst_ref=dst, send_sem=ssem, recv_sem=rsem,
                device_id=(target,), device_id_type=pl.DeviceIdType.MESH,
            )

        def compute_sub(origin, src_slot, is_b, j):
            hblk = jnp.dot(src_slot[:, :], w1b_ref[:, :],
                           preferred_element_type=jnp.float32
                           ).astype(jnp.bfloat16)
            one = jnp.bfloat16(1.0)
            hblk = hblk * (one / (one + jnp.exp(-hblk)))
            row0 = origin * m + (q if is_b else 0) + j * s
            acc_ref[pl.ds(row0, s), :] = jnp.dot(
                hblk, w2b_ref[:, :],
                preferred_element_type=jnp.float32)

        def acc_sub(b, is_b, j):
            return acc_ref[pl.ds(b * m + (q if is_b else 0) + j * s, s), :]

        barrier_sem = pltpu.get_barrier_semaphore()
        for nbr in (left, right):
            pl.semaphore_signal(
                barrier_sem, inc=1,
                device_id=(nbr,), device_id_type=pl.DeviceIdType.MESH,
            )
        pl.semaphore_wait(barrier_sem, 2)

        agcR_ref[0, :, :] = x_ref[pl.ds(0, q), :].astype(jnp.bfloat16)
        agcL_ref[0, :, :] = x_ref[pl.ds(q, q), :].astype(jnp.bfloat16)
        agR = [[None] * NSUB for _ in range(NHOP)]
        agL = [[None] * NSUB for _ in range(NHOP)]
        rsR = [[None] * NSUB for _ in range(NRS)]
        rsL = [[None] * NSUB for _ in range(NRS)]
        for j in range(NSUB):
            agR[0][j] = copy(agcR_ref.at[0, pl.ds(j * s, s)],
                             agcR_ref.at[1, pl.ds(j * s, s)],
                             agR_send.at[0, j], agR_recv.at[0, j], right)
            agR[0][j].start()
            agL[0][j] = copy(agcL_ref.at[0, pl.ds(j * s, s)],
                             agcL_ref.at[1, pl.ds(j * s, s)],
                             agL_send.at[0, j], agL_recv.at[0, j], left)
            agL[0][j].start()

        dirA = copy(agcR_ref.at[0], dirA_ref,
                    dir_sems.at[0], dir_sems.at[1], left)
        dirB = copy(agcL_ref.at[0], dirB_ref,
                    dir_sems.at[2], dir_sems.at[3], right)
        dirA.start()
        dirB.start()

        w1_copy.wait()
        w1b_ref[:, :] = w1f_ref[:, :].astype(jnp.bfloat16)
        w2_copy.wait()
        w2b_ref[:, :] = w2f_ref[:, :].astype(jnp.bfloat16)
        for j in range(NSUB):
            compute_sub(my, agcR_ref.at[0, pl.ds(j * s, s)], False, j)
            compute_sub(my, agcL_ref.at[0, pl.ds(j * s, s)], True, j)

        def ring_step(h, j, ag, agc_ref, ag_send, ag_recv,
                      rs, rsc_ref, rs_send, rs_recv, origin, is_b, target):
            sub = pl.ds(j * s, s)
            ag[h][j].wait_recv()
            if h + 1 < NHOP:
                ag[h + 1][j] = copy(agc_ref.at[h + 1, sub],
                                    agc_ref.at[h + 2, sub],
                                    ag_send.at[h + 1, j],
                                    ag_recv.at[h + 1, j], target)
                ag[h + 1][j].start()
            compute_sub(origin, agc_ref.at[h + 1, sub], is_b, j)
            if h == 0:
                rsc_ref[0, sub] = acc_sub(origin, is_b, j).astype(jnp.bfloat16)
            else:
                rs[h - 1][j].wait_recv()
                rsc_ref[h, sub] = (
                    rsc_ref[h, sub].astype(jnp.float32)
                    + acc_sub(origin, is_b, j)
                ).astype(jnp.bfloat16)
            rs[h][j] = copy(rsc_ref.at[h, sub], rsc_ref.at[h + 1, sub],
                            rs_send.at[h, j], rs_recv.at[h, j], target)
            rs[h][j].start()

        for h in range(NHOP):
            oR = (my - h - 1) % N_DEV
            oL = (my + h + 1) % N_DEV
            for j in range(NSUB):
                ring_step(h, j, agR, agcR_ref, agR_send, agR_recv,
                          rsR, rscR_ref, rsR_send, rsR_recv, oR, False, right)
                ring_step(h, j, agL, agcL_ref, agL_send, agL_recv,
                          rsL, rscL_ref, rsL_send, rsL_recv, oL, True, left)

        dirA.wait_recv()
        dirB.wait_recv()
        tA = (my + 1) % N_DEV
        tB = (my - 1) % N_DEV
        for j in range(NSUB):
            compute_sub(tA, dirA_ref.at[pl.ds(j * s, s)], False, j)
            compute_sub(tB, dirB_ref.at[pl.ds(j * s, s)], True, j)

        hL = NRS - 1
        for j in range(NSUB):
            sub = pl.ds(j * s, s)
            rsR[hL - 1][j].wait_recv()
            rscR_ref[hL, sub] = (
                rscR_ref[hL, sub].astype(jnp.float32) + acc_sub(tA, False, j)
            ).astype(jnp.bfloat16)
            rsR[hL][j] = copy(rscR_ref.at[hL, sub], rscR_ref.at[hL + 1, sub],
                              rsR_send.at[hL, j], rsR_recv.at[hL, j], right)
            rsR[hL][j].start()
            rsL[hL - 1][j].wait_recv()
            rscL_ref[hL, sub] = (
                rscL_ref[hL, sub].astype(jnp.float32) + acc_sub(tB, True, j)
            ).astype(jnp.bfloat16)
            rsL[hL][j] = copy(rscL_ref.at[hL, sub], rscL_ref.at[hL + 1, sub],
                              rsL_send.at[hL, j], rsL_recv.at[hL, j], left)
            rsL[hL][j].start()

        for j in range(NSUB):
            sub = pl.ds(j * s, s)
            rsR[hL][j].wait_recv()
            out_ref[pl.ds(j * s, s), :] = (
                rscR_ref[NRS, sub].astype(jnp.float32) + acc_sub(my, False, j)
            )
            rsL[hL][j].wait_recv()
            out_ref[pl.ds(q + j * s, s), :] = (
                rscL_ref[NRS, sub].astype(jnp.float32) + acc_sub(my, True, j)
            )

        dirA.wait_send()
        dirB.wait_send()
        for h in range(NHOP):
            for j in range(NSUB):
                agR[h][j].wait_send()
                agL[h][j].wait_send()
        for h in range(NRS):
            for j in range(NSUB):
                rsR[h][j].wait_send()
                rsL[h][j].wait_send()

    return pl.pallas_call(
        body,
        out_shape=jax.ShapeDtypeStruct((m, d), jnp.float32),
        in_specs=[
            pl.BlockSpec(memory_space=pltpu.VMEM),
            pl.BlockSpec(memory_space=pltpu.MemorySpace.HBM),
            pl.BlockSpec(memory_space=pltpu.MemorySpace.HBM),
        ],
        out_specs=pl.BlockSpec(memory_space=pltpu.VMEM),
        scratch_shapes=[
            pltpu.VMEM((NHOP + 1, q, d), jnp.bfloat16),
            pltpu.VMEM((NHOP + 1, q, d), jnp.bfloat16),
            pltpu.VMEM((q, d), jnp.bfloat16),
            pltpu.VMEM((q, d), jnp.bfloat16),
            pltpu.VMEM((d, f), jnp.float32),
            pltpu.VMEM((f, d), jnp.float32),
            pltpu.VMEM((d, f), jnp.bfloat16),
            pltpu.VMEM((f, d), jnp.bfloat16),
            pltpu.VMEM((M, d), jnp.float32),
            pltpu.VMEM((NRS + 1, q, d), jnp.bfloat16),
            pltpu.VMEM((NRS + 1, q, d), jnp.bfloat16),
            pltpu.SemaphoreType.DMA((2,)),
            pltpu.SemaphoreType.DMA((4,)),
            pltpu.SemaphoreType.DMA((NHOP, NSUB)),
            pltpu.SemaphoreType.DMA((NHOP, NSUB)),
            pltpu.SemaphoreType.DMA((NHOP, NSUB)),
            pltpu.SemaphoreType.DMA((NHOP, NSUB)),
            pltpu.SemaphoreType.DMA((NRS, NSUB)),
            pltpu.SemaphoreType.DMA((NRS, NSUB)),
            pltpu.SemaphoreType.DMA((NRS, NSUB)),
            pltpu.SemaphoreType.DMA((NRS, NSUB)),
        ],
        compiler_params=pltpu.CompilerParams(collective_id=0),
    )(x, W1, W2)
